# initial kernel scaffold (unmeasured)
import jax
import jax.numpy as jnp
from jax import lax
from jax.experimental import pallas as pl
from jax.experimental.pallas import tpu as pltpu


def kernel(
    x,
):
    def body(*refs):
        pass

    out_shape = jax.ShapeDtypeStruct(..., jnp.float32)
    return pl.pallas_call(body, out_shape=out_shape)(...)


# baseline (device time: 412219 ns/iter reference)
import jax
import jax.numpy as jnp
from jax import lax
from jax.experimental import pallas as pl
from jax.experimental.pallas import tpu as pltpu

M, N = 16384, 1024
NC = 16
CR = M // NC

_HBM = pltpu.MemorySpace.HBM


def kernel(x):
    def body(x_hbm, out_hbm, recv_vmem, xv, bv0, bv1, ov,
             lsem, osem, send_sems, recv_sems):
        my_x = lax.axis_index("x")
        my_y = lax.axis_index("y")
        my_z = lax.axis_index("z")
        partner = (my_x, my_y, 1 - my_z)
        bvs = [bv0, bv1]

        barrier = pltpu.get_barrier_semaphore()
        pl.semaphore_signal(barrier, inc=1, device_id=partner,
                            device_id_type=pl.DeviceIdType.MESH)
        pl.semaphore_wait(barrier, 1)

        rdmas = []

        def drain(c):
            sl = pl.ds(c * CR, CR)
            rdmas[c].wait_recv()
            ov[...] = (bvs[c % 2][...].astype(jnp.float32)
                       + recv_vmem[sl, :].astype(jnp.float32))
            cpo = pltpu.make_async_copy(ov, out_hbm.at[sl, :], osem)
            cpo.start()
            cpo.wait()

        for c in range(NC):
            if c >= 2:
                rdmas[c - 2].wait_send()
            sl = pl.ds(c * CR, CR)
            cp = pltpu.make_async_copy(x_hbm.at[sl, :], xv, lsem)
            cp.start()
            cp.wait()
            bvs[c % 2][...] = xv[...].astype(jnp.bfloat16)
            rdma = pltpu.make_async_remote_copy(
                src_ref=bvs[c % 2],
                dst_ref=recv_vmem.at[sl, :],
                send_sem=send_sems.at[c % 2],
                recv_sem=recv_sems.at[c],
                device_id=partner,
                device_id_type=pl.DeviceIdType.MESH,
            )
            rdma.start()
            rdmas.append(rdma)
            if c >= 1:
                drain(c - 1)

        drain(NC - 1)
        rdmas[NC - 2].wait_send()
        rdmas[NC - 1].wait_send()

    return pl.pallas_call(
        body,
        out_shape=jax.ShapeDtypeStruct((M, N), jnp.float32),
        in_specs=[pl.BlockSpec(memory_space=_HBM)],
        out_specs=pl.BlockSpec(memory_space=_HBM),
        scratch_shapes=[
            pltpu.VMEM((M, N), jnp.bfloat16),
            pltpu.VMEM((CR, N), jnp.float32),
            pltpu.VMEM((CR, N), jnp.bfloat16),
            pltpu.VMEM((CR, N), jnp.bfloat16),
            pltpu.VMEM((CR, N), jnp.float32),
            pltpu.SemaphoreType.DMA,
            pltpu.SemaphoreType.DMA,
            pltpu.SemaphoreType.DMA((2,)),
            pltpu.SemaphoreType.DMA((NC,)),
        ],
        compiler_params=pltpu.CompilerParams(
            collective_id=0, vmem_limit_bytes=56 * 1024 * 1024
        ),
    )(x)


# device time: 324626 ns/iter; 1.2698x vs baseline; 1.2698x over previous
import jax
import jax.numpy as jnp
from jax import lax
from jax.experimental import pallas as pl
from jax.experimental.pallas import tpu as pltpu

M, N = 16384, 1024
Q = M // 4
SC = 1024
NSC = Q // SC
H = Q // 2

_HBM = pltpu.MemorySpace.HBM
_MESH = pl.DeviceIdType.MESH


def kernel(x):
    def body(x_hbm, out_hbm, sq, rz, rx, ry, rd, xv, ov,
             lsem, osem, ssems, rsems):
        my_x = lax.axis_index("x")
        my_y = lax.axis_index("y")
        my_z = lax.axis_index("z")
        zp = (my_x, my_y, 1 - my_z)
        xn = (1 - my_x, my_y, my_z)
        yn = (my_x, 1 - my_y, my_z)
        q = 2 * my_x + my_y
        qx = 2 * (1 - my_x) + my_y
        qy = 2 * my_x + (1 - my_y)
        qd = 2 * (1 - my_x) + (1 - my_y)

        barrier = pltpu.get_barrier_semaphore()
        for nbr in (zp, xn, yn):
            pl.semaphore_signal(barrier, inc=1, device_id=nbr,
                                device_id_type=_MESH)
        pl.semaphore_wait(barrier, 3)

        def store_quarter(src, quarter):
            for s in range(NSC):
                sl = pl.ds(s * SC, SC)
                ov[...] = src[sl, :].astype(jnp.float32)
                cp = pltpu.make_async_copy(
                    ov, out_hbm.at[pl.ds(quarter * Q + s * SC, SC), :], osem)
                cp.start()
                cp.wait()

        for s in range(NSC):
            cp = pltpu.make_async_copy(
                x_hbm.at[pl.ds(q * Q + s * SC, SC), :], xv, lsem)
            cp.start()
            cp.wait()
            sq[pl.ds(s * SC, SC), :] = xv[...].astype(jnp.bfloat16)

        rdma_z = pltpu.make_async_remote_copy(
            src_ref=sq, dst_ref=rz,
            send_sem=ssems.at[0], recv_sem=rsems.at[0],
            device_id=zp, device_id_type=_MESH)
        rdma_z.start()
        rdma_z.wait()

        for s in range(NSC):
            sl = pl.ds(s * SC, SC)
            ov[...] = (sq[sl, :].astype(jnp.float32)
                       + rz[sl, :].astype(jnp.float32))
            cp = pltpu.make_async_copy(
                ov, out_hbm.at[pl.ds(q * Q + s * SC, SC), :], osem)
            cp.start()
            cp.wait()
            sq[sl, :] = ov[...].astype(jnp.bfloat16)

        rdma_x = pltpu.make_async_remote_copy(
            src_ref=sq, dst_ref=rx,
            send_sem=ssems.at[1], recv_sem=rsems.at[1],
            device_id=xn, device_id_type=_MESH)
        rdma_y = pltpu.make_async_remote_copy(
            src_ref=sq, dst_ref=ry,
            send_sem=ssems.at[2], recv_sem=rsems.at[2],
            device_id=yn, device_id_type=_MESH)
        rdma_x.start()
        rdma_y.start()
        rdma_x.wait()
        rdma_y.wait()

        store_quarter(rx, qx)
        store_quarter(ry, qy)

        rdma_dx = pltpu.make_async_remote_copy(
            src_ref=ry.at[pl.ds(0, H), :], dst_ref=rd.at[pl.ds(0, H), :],
            send_sem=ssems.at[3], recv_sem=rsems.at[3],
            device_id=xn, device_id_type=_MESH)
        rdma_dy = pltpu.make_async_remote_copy(
            src_ref=rx.at[pl.ds(H, H), :], dst_ref=rd.at[pl.ds(H, H), :],
            send_sem=ssems.at[4], recv_sem=rsems.at[4],
            device_id=yn, device_id_type=_MESH)
        rdma_dx.start()
        rdma_dy.start()
        rdma_dx.wait()
        rdma_dy.wait()

        store_quarter(rd, qd)

    return pl.pallas_call(
        body,
        out_shape=jax.ShapeDtypeStruct((M, N), jnp.float32),
        in_specs=[pl.BlockSpec(memory_space=_HBM)],
        out_specs=pl.BlockSpec(memory_space=_HBM),
        scratch_shapes=[
            pltpu.VMEM((Q, N), jnp.bfloat16),
            pltpu.VMEM((Q, N), jnp.bfloat16),
            pltpu.VMEM((Q, N), jnp.bfloat16),
            pltpu.VMEM((Q, N), jnp.bfloat16),
            pltpu.VMEM((Q, N), jnp.bfloat16),
            pltpu.VMEM((SC, N), jnp.float32),
            pltpu.VMEM((SC, N), jnp.float32),
            pltpu.SemaphoreType.DMA,
            pltpu.SemaphoreType.DMA,
            pltpu.SemaphoreType.DMA((5,)),
            pltpu.SemaphoreType.DMA((5,)),
        ],
        compiler_params=pltpu.CompilerParams(
            collective_id=0, vmem_limit_bytes=56 * 1024 * 1024
        ),
    )(x)


# device time: 217202 ns/iter; 1.8979x vs baseline; 1.4946x over previous
import jax
import jax.numpy as jnp
from jax import lax
from jax.experimental import pallas as pl
from jax.experimental.pallas import tpu as pltpu

M, N = 16384, 1024
Q = M // 4
NP = 4
PR = Q // NP

_HBM = pltpu.MemorySpace.HBM
_MESH = pl.DeviceIdType.MESH


def kernel(x):
    def body(x_hbm, out_hbm, sq, rz, rx, ry, rd, xv, ov, lsem, osem,
             sz, rzs, sx1, rx1, sy1, ry1, sx2, rx2, sy2, ry2):
        my_x = lax.axis_index("x")
        my_y = lax.axis_index("y")
        my_z = lax.axis_index("z")
        zp = (my_x, my_y, 1 - my_z)
        xn = (1 - my_x, my_y, my_z)
        yn = (my_x, 1 - my_y, my_z)
        q = 2 * my_x + my_y
        qx = 2 * (1 - my_x) + my_y
        qy = 2 * my_x + (1 - my_y)
        qd = 2 * (1 - my_x) + (1 - my_y)

        barrier = pltpu.get_barrier_semaphore()
        for nbr in (zp, xn, yn):
            pl.semaphore_signal(barrier, inc=1, device_id=nbr,
                                device_id_type=_MESH)
        pl.semaphore_wait(barrier, 3)

        def piece(buf, p):
            return buf.at[pl.ds(p * PR, PR), :]

        def store_piece(src, p, quarter):
            ov[...] = src[pl.ds(p * PR, PR), :].astype(jnp.float32)
            cp = pltpu.make_async_copy(
                ov, out_hbm.at[pl.ds(quarter * Q + p * PR, PR), :], osem)
            cp.start()
            cp.wait()

        z_rdmas = []
        for p in range(NP):
            cp = pltpu.make_async_copy(
                x_hbm.at[pl.ds(q * Q + p * PR, PR), :], xv, lsem)
            cp.start()
            cp.wait()
            sq[pl.ds(p * PR, PR), :] = xv[...].astype(jnp.bfloat16)
            rdma = pltpu.make_async_remote_copy(
                src_ref=piece(sq, p), dst_ref=piece(rz, p),
                send_sem=sz.at[p], recv_sem=rzs.at[p],
                device_id=zp, device_id_type=_MESH)
            rdma.start()
            z_rdmas.append(rdma)

        r1x_rdmas, r1y_rdmas = [], []
        for p in range(NP):
            z_rdmas[p].wait_send()
            z_rdmas[p].wait_recv()
            sl = pl.ds(p * PR, PR)
            ov[...] = (sq[sl, :].astype(jnp.float32)
                       + rz[sl, :].astype(jnp.float32))
            sq[sl, :] = ov[...].astype(jnp.bfloat16)
            rdma = pltpu.make_async_remote_copy(
                src_ref=piece(sq, p), dst_ref=piece(rx, p),
                send_sem=sx1.at[p], recv_sem=rx1.at[p],
                device_id=xn, device_id_type=_MESH)
            rdma.start()
            r1x_rdmas.append(rdma)
            rdma = pltpu.make_async_remote_copy(
                src_ref=piece(sq, p), dst_ref=piece(ry, p),
                send_sem=sy1.at[p], recv_sem=ry1.at[p],
                device_id=yn, device_id_type=_MESH)
            rdma.start()
            r1y_rdmas.append(rdma)
            cp = pltpu.make_async_copy(
                ov, out_hbm.at[pl.ds(q * Q + p * PR, PR), :], osem)
            cp.start()
            cp.wait()

        r2_rdmas = []
        for p in range(NP):
            r1x_rdmas[p].wait_recv()
            store_piece(rx, p, qx)
            if p % 2 == 1:
                rdma = pltpu.make_async_remote_copy(
                    src_ref=piece(rx, p), dst_ref=piece(rd, p),
                    send_sem=sy2.at[p], recv_sem=ry2.at[p],
                    device_id=yn, device_id_type=_MESH)
                rdma.start()
                r2_rdmas.append(rdma)
            r1y_rdmas[p].wait_recv()
            store_piece(ry, p, qy)
            if p % 2 == 0:
                rdma = pltpu.make_async_remote_copy(
                    src_ref=piece(ry, p), dst_ref=piece(rd, p),
                    send_sem=sx2.at[p], recv_sem=rx2.at[p],
                    device_id=xn, device_id_type=_MESH)
                rdma.start()
                r2_rdmas.append(rdma)

        for p in range(NP):
            rdma = pltpu.make_async_remote_copy(
                src_ref=piece(ry, p), dst_ref=piece(rd, p),
                send_sem=sx2.at[p] if p % 2 == 0 else sy2.at[p],
                recv_sem=rx2.at[p] if p % 2 == 0 else ry2.at[p],
                device_id=xn if p % 2 == 0 else yn,
                device_id_type=_MESH)
            rdma.wait_recv()
            store_piece(rd, p, qd)

        for rdma in r1x_rdmas + r1y_rdmas + r2_rdmas:
            rdma.wait_send()

    return pl.pallas_call(
        body,
        out_shape=jax.ShapeDtypeStruct((M, N), jnp.float32),
        in_specs=[pl.BlockSpec(memory_space=_HBM)],
        out_specs=pl.BlockSpec(memory_space=_HBM),
        scratch_shapes=[
            pltpu.VMEM((Q, N), jnp.bfloat16),
            pltpu.VMEM((Q, N), jnp.bfloat16),
            pltpu.VMEM((Q, N), jnp.bfloat16),
            pltpu.VMEM((Q, N), jnp.bfloat16),
            pltpu.VMEM((Q, N), jnp.bfloat16),
            pltpu.VMEM((PR, N), jnp.float32),
            pltpu.VMEM((PR, N), jnp.float32),
            pltpu.SemaphoreType.DMA,
            pltpu.SemaphoreType.DMA,
            pltpu.SemaphoreType.DMA((NP,)),
            pltpu.SemaphoreType.DMA((NP,)),
            pltpu.SemaphoreType.DMA((NP,)),
            pltpu.SemaphoreType.DMA((NP,)),
            pltpu.SemaphoreType.DMA((NP,)),
            pltpu.SemaphoreType.DMA((NP,)),
            pltpu.SemaphoreType.DMA((NP,)),
            pltpu.SemaphoreType.DMA((NP,)),
            pltpu.SemaphoreType.DMA((NP,)),
            pltpu.SemaphoreType.DMA((NP,)),
        ],
        compiler_params=pltpu.CompilerParams(
            collective_id=0, vmem_limit_bytes=56 * 1024 * 1024
        ),
    )(x)


# device time: 202042 ns/iter; 2.0403x vs baseline; 1.0750x over previous
import jax
import jax.numpy as jnp
from jax import lax
from jax.experimental import pallas as pl
from jax.experimental.pallas import tpu as pltpu

M, N = 16384, 1024
Q = M // 4
NP = 8
PR = Q // NP

_HBM = pltpu.MemorySpace.HBM
_MESH = pl.DeviceIdType.MESH


def kernel(x):
    def body(x_hbm, out_hbm, sq, rz, rx, ry, rd, xv0, xv1, ov0, ov1,
             lsems, osems, sz, rzs, sx1, rx1, sy1, ry1, sx2, rx2,
             sy2, ry2):
        my_x = lax.axis_index("x")
        my_y = lax.axis_index("y")
        my_z = lax.axis_index("z")
        zp = (my_x, my_y, 1 - my_z)
        xn = (1 - my_x, my_y, my_z)
        yn = (my_x, 1 - my_y, my_z)
        q = 2 * my_x + my_y
        qx = 2 * (1 - my_x) + my_y
        qy = 2 * my_x + (1 - my_y)
        qd = 2 * (1 - my_x) + (1 - my_y)
        xvs = [xv0, xv1]
        ovs = [ov0, ov1]

        barrier = pltpu.get_barrier_semaphore()
        for nbr in (zp, xn, yn):
            pl.semaphore_signal(barrier, inc=1, device_id=nbr,
                                device_id_type=_MESH)
        pl.semaphore_wait(barrier, 3)

        def piece(buf, p):
            return buf.at[pl.ds(p * PR, PR), :]

        out_pending = [None, None]
        store_ct = [0]

        def acquire():
            i = store_ct[0] % 2
            store_ct[0] += 1
            if out_pending[i] is not None:
                out_pending[i].wait()
            return i

        def commit(i, quarter, p):
            cp = pltpu.make_async_copy(
                ovs[i], out_hbm.at[pl.ds(quarter * Q + p * PR, PR), :],
                osems.at[i])
            cp.start()
            out_pending[i] = cp

        def store_piece(src, p, quarter):
            i = acquire()
            ovs[i][...] = src[pl.ds(p * PR, PR), :].astype(jnp.float32)
            commit(i, quarter, p)

        def in_dma(p):
            cp = pltpu.make_async_copy(
                x_hbm.at[pl.ds(q * Q + p * PR, PR), :], xvs[p % 2],
                lsems.at[p % 2])
            cp.start()
            return cp

        z_rdmas = []
        in_pending = in_dma(0)
        for p in range(NP):
            in_pending.wait()
            if p + 1 < NP:
                in_pending = in_dma(p + 1)
            sq[pl.ds(p * PR, PR), :] = xvs[p % 2][...].astype(jnp.bfloat16)
            rdma = pltpu.make_async_remote_copy(
                src_ref=piece(sq, p), dst_ref=piece(rz, p),
                send_sem=sz.at[p], recv_sem=rzs.at[p],
                device_id=zp, device_id_type=_MESH)
            rdma.start()
            z_rdmas.append(rdma)

        r1x_rdmas, r1y_rdmas = [], []
        for p in range(NP):
            z_rdmas[p].wait_send()
            z_rdmas[p].wait_recv()
            sl = pl.ds(p * PR, PR)
            i = acquire()
            ovs[i][...] = (sq[sl, :].astype(jnp.float32)
                           + rz[sl, :].astype(jnp.float32))
            sq[sl, :] = ovs[i][...].astype(jnp.bfloat16)
            rdma = pltpu.make_async_remote_copy(
                src_ref=piece(sq, p), dst_ref=piece(rx, p),
                send_sem=sx1.at[p], recv_sem=rx1.at[p],
                device_id=xn, device_id_type=_MESH)
            rdma.start()
            r1x_rdmas.append(rdma)
            rdma = pltpu.make_async_remote_copy(
                src_ref=piece(sq, p), dst_ref=piece(ry, p),
                send_sem=sy1.at[p], recv_sem=ry1.at[p],
                device_id=yn, device_id_type=_MESH)
            rdma.start()
            r1y_rdmas.append(rdma)
            commit(i, q, p)

        r2_rdmas = []
        for p in range(NP):
            r1y_rdmas[p].wait_recv()
            if p % 2 == 0:
                rdma = pltpu.make_async_remote_copy(
                    src_ref=piece(ry, p), dst_ref=piece(rd, p),
                    send_sem=sx2.at[p], recv_sem=rx2.at[p],
                    device_id=xn, device_id_type=_MESH)
                rdma.start()
                r2_rdmas.append(rdma)
            store_piece(ry, p, qy)
            r1x_rdmas[p].wait_recv()
            if p % 2 == 1:
                rdma = pltpu.make_async_remote_copy(
                    src_ref=piece(rx, p), dst_ref=piece(rd, p),
                    send_sem=sy2.at[p], recv_sem=ry2.at[p],
                    device_id=yn, device_id_type=_MESH)
                rdma.start()
                r2_rdmas.append(rdma)
            store_piece(rx, p, qx)

        for p in range(NP):
            rdma = pltpu.make_async_remote_copy(
                src_ref=piece(ry, p), dst_ref=piece(rd, p),
                send_sem=sx2.at[p] if p % 2 == 0 else sy2.at[p],
                recv_sem=rx2.at[p] if p % 2 == 0 else ry2.at[p],
                device_id=xn if p % 2 == 0 else yn,
                device_id_type=_MESH)
            rdma.wait_recv()
            store_piece(rd, p, qd)

        for rdma in r1x_rdmas + r1y_rdmas + r2_rdmas:
            rdma.wait_send()
        for cp in out_pending:
            if cp is not None:
                cp.wait()

    return pl.pallas_call(
        body,
        out_shape=jax.ShapeDtypeStruct((M, N), jnp.float32),
        in_specs=[pl.BlockSpec(memory_space=_HBM)],
        out_specs=pl.BlockSpec(memory_space=_HBM),
        scratch_shapes=[
            pltpu.VMEM((Q, N), jnp.bfloat16),
            pltpu.VMEM((Q, N), jnp.bfloat16),
            pltpu.VMEM((Q, N), jnp.bfloat16),
            pltpu.VMEM((Q, N), jnp.bfloat16),
            pltpu.VMEM((Q, N), jnp.bfloat16),
            pltpu.VMEM((PR, N), jnp.float32),
            pltpu.VMEM((PR, N), jnp.float32),
            pltpu.VMEM((PR, N), jnp.float32),
            pltpu.VMEM((PR, N), jnp.float32),
            pltpu.SemaphoreType.DMA((2,)),
            pltpu.SemaphoreType.DMA((2,)),
            pltpu.SemaphoreType.DMA((NP,)),
            pltpu.SemaphoreType.DMA((NP,)),
            pltpu.SemaphoreType.DMA((NP,)),
            pltpu.SemaphoreType.DMA((NP,)),
            pltpu.SemaphoreType.DMA((NP,)),
            pltpu.SemaphoreType.DMA((NP,)),
            pltpu.SemaphoreType.DMA((NP,)),
            pltpu.SemaphoreType.DMA((NP,)),
            pltpu.SemaphoreType.DMA((NP,)),
            pltpu.SemaphoreType.DMA((NP,)),
        ],
        compiler_params=pltpu.CompilerParams(
            collective_id=0, vmem_limit_bytes=56 * 1024 * 1024
        ),
    )(x)
